# baseline (speedup 1.0000x reference)
import functools

import jax
import jax.numpy as jnp
from jax.experimental import pallas as pl
from jax.experimental.pallas import tpu as pltpu

_N = 19
_S = 24
_P = 5
_GCHUNK = 128


def _gcn_body(g_total, x_ref, w_ref, mdT_ref, ms_ref, md_ref, k_ref,
              kloop_ref, w1_ref, b1_ref, w2_ref, b2_ref, o_ref,
              abd_ref, x5_ref, zbuf_ref, hbuf_ref, z2buf_ref):
    n = _N
    g = _GCHUNK
    sizes = [_P] * (g // _P)
    if g % _P:
        sizes.append(g % _P)
    ngrp = len(sizes)
    rows = _P * _S

    w = w_ref[...]

    deg = jnp.dot(w, mdT_ref[...],
                  preferred_element_type=jnp.float32) + 1.0
    dis = jnp.where(deg > 0,
                    jax.lax.rsqrt(jnp.maximum(deg, 1e-12)),
                    0.0)
    dis_s = jnp.dot(dis, ms_ref[...],
                    preferred_element_type=jnp.float32)
    dis_d = jnp.dot(dis, md_ref[...],
                    preferred_element_type=jnp.float32)
    norm = dis_s * w * dis_d

    a_flat = (jnp.dot(norm, k_ref[...],
                      preferred_element_type=jnp.float32)
              + jnp.dot(dis * dis, kloop_ref[...],
                        preferred_element_type=jnp.float32))

    @pl.when(pl.program_id(0) == 0)
    def _init():
        abd_ref[...] = jnp.zeros(abd_ref.shape, dtype=jnp.float32)
        x5_ref[...] = jnp.zeros(x5_ref.shape, dtype=jnp.float32)

    x = x_ref[...]
    off = 0
    for t, p in enumerate(sizes):
        a3 = a_flat[off:off + p].reshape(p, n, n)
        for q in range(p):
            abd_ref[t, _S * q:_S * q + n, _S * q:_S * q + n] = a3[q]
            x5_ref[t, _S * q:_S * q + n, :] = x[off + q]
        off += p

    w1 = w1_ref[...]
    b1 = b1_ref[...]
    w2 = w2_ref[...]
    b2 = b2_ref[...]
    for t, p in enumerate(sizes):
        r = p * _S
        zbuf_ref[rows * t:rows * t + r, :] = jnp.dot(
            abd_ref[t, 0:r, 0:r], x5_ref[t, 0:r, :],
            preferred_element_type=jnp.float32)
    hbuf_ref[...] = jax.nn.sigmoid(
        jnp.dot(zbuf_ref[...], w1, preferred_element_type=jnp.float32) + b1)
    for t, p in enumerate(sizes):
        r = p * _S
        z2buf_ref[rows * t:rows * t + r, :] = jnp.dot(
            abd_ref[t, 0:r, 0:r], hbuf_ref[rows * t:rows * t + r, :],
            preferred_element_type=jnp.float32)
    tt = o_ref.shape[1]
    off = 0
    for t, p in enumerate(sizes):
        r = p * _S
        y = jax.nn.sigmoid(
            jnp.dot(z2buf_ref[rows * t:rows * t + r, :], w2,
                    preferred_element_type=jnp.float32) + b2)
        for q in range(p):
            u, v = divmod(off + q, tt)
            o_ref[u, v, :, :] = y[_S * q:_S * q + n, :]
        off += p


@functools.partial(jax.jit, static_argnames=("bb", "tt", "interpret"))
def _run(x3d, ew, mdT, ms, md, kmat, kloop, W1, b1, W2, b2, bb, tt,
         interpret=False):
    n = _N
    g_total = ew.shape[0]
    d_in = x3d.shape[2]
    d_out = W2.shape[1]
    grid = (g_total + _GCHUNK - 1) // _GCHUNK
    ngrp = (_GCHUNK + _P - 1) // _P
    bblk = _GCHUNK // tt

    out = pl.pallas_call(
        functools.partial(_gcn_body, g_total),
        grid=(grid,),
        in_specs=[
            pl.BlockSpec((_GCHUNK, n, d_in), lambda i: (i, 0, 0)),
            pl.BlockSpec((_GCHUNK, ew.shape[1]), lambda i: (i, 0)),
            pl.BlockSpec(mdT.shape, lambda i: (0, 0)),
            pl.BlockSpec(ms.shape, lambda i: (0, 0)),
            pl.BlockSpec(md.shape, lambda i: (0, 0)),
            pl.BlockSpec(kmat.shape, lambda i: (0, 0)),
            pl.BlockSpec(kloop.shape, lambda i: (0, 0)),
            pl.BlockSpec(W1.shape, lambda i: (0, 0)),
            pl.BlockSpec(b1.shape, lambda i: (0, 0)),
            pl.BlockSpec(W2.shape, lambda i: (0, 0)),
            pl.BlockSpec(b2.shape, lambda i: (0, 0)),
        ],
        out_specs=pl.BlockSpec((bblk, tt, n, d_out), lambda i: (i, 0, 0, 0)),
        out_shape=jax.ShapeDtypeStruct((bb, tt, n, d_out), jnp.float32),
        scratch_shapes=[
            pltpu.VMEM((ngrp, _P * _S, _P * _S), jnp.float32),
            pltpu.VMEM((ngrp, _P * _S, d_in), jnp.float32),
            pltpu.VMEM((ngrp * _P * _S, d_in), jnp.float32),
            pltpu.VMEM((ngrp * _P * _S, W1.shape[1]), jnp.float32),
            pltpu.VMEM((ngrp * _P * _S, W1.shape[1]), jnp.float32),
        ],
        compiler_params=pltpu.CompilerParams(
            dimension_semantics=("arbitrary",)),
        interpret=interpret,
    )(x3d, ew, mdT, ms, md, kmat, kloop, W1, b1, W2, b2)
    return out


def kernel(feature_all, graph_index, graph_weight, W1, b1, W2, b2):
    Bb, Tt, n, d_in = feature_all.shape
    g_total = Bb * Tt
    x3d = feature_all.reshape(g_total, n, d_in)
    ew = graph_weight.reshape(g_total, -1)

    src = graph_index[0, 0]
    dst = graph_index[0, 1]
    msT = jax.nn.one_hot(src, n, dtype=jnp.float32)
    mdT = jax.nn.one_hot(dst, n, dtype=jnp.float32)
    kmat = (mdT[:, :, None] * msT[:, None, :]).reshape(src.shape[0], n * n)
    kloop = (jnp.eye(n, dtype=jnp.float32)[:, :, None]
             * jnp.eye(n, dtype=jnp.float32)[:, None, :]).reshape(n, n * n)

    return _run(x3d, ew, mdT, msT.T, mdT.T, kmat, kloop,
                W1, b1.reshape(1, -1), W2, b2.reshape(1, -1),
                bb=Bb, tt=Tt)


# speedup vs baseline: 1.1024x; 1.1024x over previous
import functools

import jax
import jax.numpy as jnp
from jax.experimental import pallas as pl
from jax.experimental.pallas import tpu as pltpu

_N = 19
_S = 24
_P = 5
_GCHUNK = 240


def _gcn_body(g_total, x_ref, w_ref, mdT_ref, ms_ref, md_ref, k_ref,
              kloop_ref, w1_ref, b1_ref, w2_ref, b2_ref, o_ref,
              abd_ref, x5_ref, zbuf_ref, hbuf_ref, z2buf_ref):
    n = _N
    g = _GCHUNK
    ngrp = g // _P
    rows = _P * _S

    valid = g_total - pl.program_id(0) * g
    gmask = (jax.lax.broadcasted_iota(jnp.int32, (g, 1), 0) < valid)
    w = jnp.where(gmask, w_ref[...], 0.0)

    deg = jnp.dot(w, mdT_ref[...],
                  preferred_element_type=jnp.float32) + 1.0
    dis = jnp.where(deg > 0,
                    jax.lax.rsqrt(jnp.maximum(deg, 1e-12)),
                    0.0)
    dis_s = jnp.dot(dis, ms_ref[...],
                    preferred_element_type=jnp.float32)
    dis_d = jnp.dot(dis, md_ref[...],
                    preferred_element_type=jnp.float32)
    norm = dis_s * w * dis_d

    a_flat = (jnp.dot(norm, k_ref[...],
                      preferred_element_type=jnp.float32)
              + jnp.dot(dis * dis, kloop_ref[...],
                        preferred_element_type=jnp.float32))
    a4 = a_flat.reshape(ngrp, _P, n, n)

    @pl.when(pl.program_id(0) == 0)
    def _init():
        abd_ref[...] = jnp.zeros(abd_ref.shape, dtype=jnp.float32)
        zpad = jnp.zeros((ngrp, _S - n, x_ref.shape[2]), dtype=jnp.float32)
        for q in range(_P):
            x5_ref[:, _S * q + n:_S * (q + 1), :] = zpad

    gmask3 = gmask[:, :, None]
    x = jnp.where(gmask3, x_ref[...], 0.0)
    x4 = x.reshape(ngrp, _P, n, x_ref.shape[2])
    for q in range(_P):
        abd_ref[:, _S * q:_S * q + n, _S * q:_S * q + n] = a4[:, q]
        x5_ref[:, _S * q:_S * q + n, :] = x4[:, q]

    w1 = w1_ref[...]
    b1 = b1_ref[...]
    w2 = w2_ref[...]
    b2 = b2_ref[...]
    for t in range(ngrp):
        zbuf_ref[rows * t:rows * (t + 1), :] = jnp.dot(
            abd_ref[t], x5_ref[t], preferred_element_type=jnp.float32)
    hbuf_ref[...] = jax.nn.sigmoid(
        jnp.dot(zbuf_ref[...], w1, preferred_element_type=jnp.float32) + b1)
    for t in range(ngrp):
        z2buf_ref[rows * t:rows * (t + 1), :] = jnp.dot(
            abd_ref[t], hbuf_ref[rows * t:rows * (t + 1), :],
            preferred_element_type=jnp.float32)
    for t in range(ngrp):
        y = jax.nn.sigmoid(
            jnp.dot(z2buf_ref[rows * t:rows * (t + 1), :], w2,
                    preferred_element_type=jnp.float32) + b2)
        for q in range(_P):
            o_ref[_P * t + q, :, :] = y[_S * q:_S * q + n, :]


@functools.partial(jax.jit, static_argnames=("interpret",))
def _run(x3d, ew, mdT, ms, md, kmat, kloop, W1, b1, W2, b2, interpret=False):
    n = _N
    g_total = ew.shape[0]
    d_in = x3d.shape[2]
    d_out = W2.shape[1]
    grid = (g_total + _GCHUNK - 1) // _GCHUNK
    ngrp = _GCHUNK // _P

    out = pl.pallas_call(
        functools.partial(_gcn_body, g_total),
        grid=(grid,),
        in_specs=[
            pl.BlockSpec((_GCHUNK, n, d_in), lambda i: (i, 0, 0)),
            pl.BlockSpec((_GCHUNK, ew.shape[1]), lambda i: (i, 0)),
            pl.BlockSpec(mdT.shape, lambda i: (0, 0)),
            pl.BlockSpec(ms.shape, lambda i: (0, 0)),
            pl.BlockSpec(md.shape, lambda i: (0, 0)),
            pl.BlockSpec(kmat.shape, lambda i: (0, 0)),
            pl.BlockSpec(kloop.shape, lambda i: (0, 0)),
            pl.BlockSpec(W1.shape, lambda i: (0, 0)),
            pl.BlockSpec(b1.shape, lambda i: (0, 0)),
            pl.BlockSpec(W2.shape, lambda i: (0, 0)),
            pl.BlockSpec(b2.shape, lambda i: (0, 0)),
        ],
        out_specs=pl.BlockSpec((_GCHUNK, n, d_out), lambda i: (i, 0, 0)),
        out_shape=jax.ShapeDtypeStruct((g_total, n, d_out), jnp.float32),
        scratch_shapes=[
            pltpu.VMEM((ngrp, _P * _S, _P * _S), jnp.float32),
            pltpu.VMEM((ngrp, _P * _S, d_in), jnp.float32),
            pltpu.VMEM((ngrp * _P * _S, d_in), jnp.float32),
            pltpu.VMEM((ngrp * _P * _S, W1.shape[1]), jnp.float32),
            pltpu.VMEM((ngrp * _P * _S, W1.shape[1]), jnp.float32),
        ],
        compiler_params=pltpu.CompilerParams(
            dimension_semantics=("arbitrary",)),
        interpret=interpret,
    )(x3d, ew, mdT, ms, md, kmat, kloop, W1, b1, W2, b2)
    return out


def kernel(feature_all, graph_index, graph_weight, W1, b1, W2, b2):
    Bb, Tt, n, d_in = feature_all.shape
    g_total = Bb * Tt
    x3d = feature_all.reshape(g_total, n, d_in)
    ew = graph_weight.reshape(g_total, -1)

    src = graph_index[0, 0]
    dst = graph_index[0, 1]
    msT = jax.nn.one_hot(src, n, dtype=jnp.float32)
    mdT = jax.nn.one_hot(dst, n, dtype=jnp.float32)
    kmat = (mdT[:, :, None] * msT[:, None, :]).reshape(src.shape[0], n * n)
    kloop = (jnp.eye(n, dtype=jnp.float32)[:, :, None]
             * jnp.eye(n, dtype=jnp.float32)[:, None, :]).reshape(n, n * n)

    out = _run(x3d, ew, mdT, msT.T, mdT.T, kmat, kloop,
               W1, b1.reshape(1, -1), W2, b2.reshape(1, -1))
    return out.reshape(Bb, Tt, n, W2.shape[1])


# speedup vs baseline: 1.1133x; 1.0099x over previous
import functools

import jax
import jax.numpy as jnp
from jax.experimental import pallas as pl
from jax.experimental.pallas import tpu as pltpu

_N = 19
_S = 24
_P = 5
_GCHUNK = 320


def _gcn_body(g_total, x_ref, w_ref, mdT_ref, ms_ref, md_ref, k_ref,
              kloop_ref, w1_ref, b1_ref, w2_ref, b2_ref, o_ref,
              abd_ref, x5_ref, zbuf_ref, hbuf_ref, z2buf_ref):
    n = _N
    g = _GCHUNK
    ngrp = g // _P
    rows = _P * _S

    valid = g_total - pl.program_id(0) * g
    gmask = (jax.lax.broadcasted_iota(jnp.int32, (g, 1), 0) < valid)
    w = jnp.where(gmask, w_ref[...], 0.0)

    deg = jnp.dot(w, mdT_ref[...],
                  preferred_element_type=jnp.float32) + 1.0
    dis = jnp.where(deg > 0,
                    jax.lax.rsqrt(jnp.maximum(deg, 1e-12)),
                    0.0)
    dis_s = jnp.dot(dis, ms_ref[...],
                    preferred_element_type=jnp.float32)
    dis_d = jnp.dot(dis, md_ref[...],
                    preferred_element_type=jnp.float32)
    norm = dis_s * w * dis_d

    a_flat = (jnp.dot(norm, k_ref[...],
                      preferred_element_type=jnp.float32)
              + jnp.dot(dis * dis, kloop_ref[...],
                        preferred_element_type=jnp.float32))
    a4 = a_flat.reshape(ngrp, _P, n, n)

    @pl.when(pl.program_id(0) == 0)
    def _init():
        abd_ref[...] = jnp.zeros(abd_ref.shape, dtype=jnp.float32)
        zpad = jnp.zeros((ngrp, _S - n, x_ref.shape[2]), dtype=jnp.float32)
        for q in range(_P):
            x5_ref[:, _S * q + n:_S * (q + 1), :] = zpad

    gmask3 = gmask[:, :, None]
    x = jnp.where(gmask3, x_ref[...], 0.0)
    x4 = x.reshape(ngrp, _P, n, x_ref.shape[2])
    for q in range(_P):
        abd_ref[:, _S * q:_S * q + n, _S * q:_S * q + n] = a4[:, q]
        x5_ref[:, _S * q:_S * q + n, :] = x4[:, q]

    w1 = w1_ref[...]
    b1 = b1_ref[...]
    w2 = w2_ref[...]
    b2 = b2_ref[...]
    for t in range(ngrp):
        zbuf_ref[rows * t:rows * (t + 1), :] = jnp.dot(
            abd_ref[t], x5_ref[t], preferred_element_type=jnp.float32)
    hbuf_ref[...] = jax.nn.sigmoid(
        jnp.dot(zbuf_ref[...], w1, preferred_element_type=jnp.float32) + b1)
    for t in range(ngrp):
        z2buf_ref[rows * t:rows * (t + 1), :] = jnp.dot(
            abd_ref[t], hbuf_ref[rows * t:rows * (t + 1), :],
            preferred_element_type=jnp.float32)
    for t in range(ngrp):
        y = jax.nn.sigmoid(
            jnp.dot(z2buf_ref[rows * t:rows * (t + 1), :], w2,
                    preferred_element_type=jnp.float32) + b2)
        for q in range(_P):
            o_ref[_P * t + q, :, :] = y[_S * q:_S * q + n, :]


@functools.partial(jax.jit, static_argnames=("interpret",))
def _run(x3d, ew, mdT, ms, md, kmat, kloop, W1, b1, W2, b2, interpret=False):
    n = _N
    g_total = ew.shape[0]
    d_in = x3d.shape[2]
    d_out = W2.shape[1]
    grid = (g_total + _GCHUNK - 1) // _GCHUNK
    ngrp = _GCHUNK // _P

    out = pl.pallas_call(
        functools.partial(_gcn_body, g_total),
        grid=(grid,),
        in_specs=[
            pl.BlockSpec((_GCHUNK, n, d_in), lambda i: (i, 0, 0)),
            pl.BlockSpec((_GCHUNK, ew.shape[1]), lambda i: (i, 0)),
            pl.BlockSpec(mdT.shape, lambda i: (0, 0)),
            pl.BlockSpec(ms.shape, lambda i: (0, 0)),
            pl.BlockSpec(md.shape, lambda i: (0, 0)),
            pl.BlockSpec(kmat.shape, lambda i: (0, 0)),
            pl.BlockSpec(kloop.shape, lambda i: (0, 0)),
            pl.BlockSpec(W1.shape, lambda i: (0, 0)),
            pl.BlockSpec(b1.shape, lambda i: (0, 0)),
            pl.BlockSpec(W2.shape, lambda i: (0, 0)),
            pl.BlockSpec(b2.shape, lambda i: (0, 0)),
        ],
        out_specs=pl.BlockSpec((_GCHUNK, n, d_out), lambda i: (i, 0, 0)),
        out_shape=jax.ShapeDtypeStruct((g_total, n, d_out), jnp.float32),
        scratch_shapes=[
            pltpu.VMEM((ngrp, _P * _S, _P * _S), jnp.float32),
            pltpu.VMEM((ngrp, _P * _S, d_in), jnp.float32),
            pltpu.VMEM((ngrp * _P * _S, d_in), jnp.float32),
            pltpu.VMEM((ngrp * _P * _S, W1.shape[1]), jnp.float32),
            pltpu.VMEM((ngrp * _P * _S, W1.shape[1]), jnp.float32),
        ],
        compiler_params=pltpu.CompilerParams(
            dimension_semantics=("arbitrary",)),
        interpret=interpret,
    )(x3d, ew, mdT, ms, md, kmat, kloop, W1, b1, W2, b2)
    return out


def kernel(feature_all, graph_index, graph_weight, W1, b1, W2, b2):
    Bb, Tt, n, d_in = feature_all.shape
    g_total = Bb * Tt
    x3d = feature_all.reshape(g_total, n, d_in)
    ew = graph_weight.reshape(g_total, -1)

    src = graph_index[0, 0]
    dst = graph_index[0, 1]
    msT = jax.nn.one_hot(src, n, dtype=jnp.float32)
    mdT = jax.nn.one_hot(dst, n, dtype=jnp.float32)
    kmat = (mdT[:, :, None] * msT[:, None, :]).reshape(src.shape[0], n * n)
    kloop = (jnp.eye(n, dtype=jnp.float32)[:, :, None]
             * jnp.eye(n, dtype=jnp.float32)[:, None, :]).reshape(n, n * n)

    out = _run(x3d, ew, mdT, msT.T, mdT.T, kmat, kloop,
               W1, b1.reshape(1, -1), W2, b2.reshape(1, -1))
    return out.reshape(Bb, Tt, n, W2.shape[1])
